# baseline (speedup 1.0000x reference)
import functools

import jax
import jax.numpy as jnp
from jax import lax
from jax.experimental import pallas as pl
from jax.experimental.pallas import tpu as pltpu
from jax.experimental.pallas import tpu_sc as plsc

M_TILE = 1024
N_CHUNKS = 1
K = 8
NW = 32
L = 16
EP = 128


def _scores_body(x_ref, w1_ref, b1_ref, w2_ref, b2_ref, temp_ref,
                 wpad_ref, w_ref):
    hid = jnp.dot(x_ref[...], w1_ref[...], preferred_element_type=jnp.float32)
    hid = hid + b1_ref[...]
    hid = hid * 0.5 * (1.0 + jax.lax.erf(hid * 0.7071067811865476))
    scores = jnp.dot(hid, w2_ref[...], preferred_element_type=jnp.float32)
    logits = (scores + b2_ref[...]) / temp_ref[0, 0]
    lmax = jnp.max(logits, axis=-1, keepdims=True)
    ex = jnp.exp(logits - lmax)
    w = ex / jnp.sum(ex, axis=-1, keepdims=True)
    w_ref[...] = w
    mt = w.shape[0]
    pad = jnp.zeros((mt, EP - w.shape[1]), jnp.float32)
    wpad_ref[...] = jnp.concatenate([w, pad], axis=1)


def _tc_scores(xf, W1, b1r, W2, b2r, tr):
    mc, D = xf.shape
    Dh, E = W2.shape
    mt = min(M_TILE, mc)
    return pl.pallas_call(
        _scores_body,
        grid=(mc // mt,),
        in_specs=[
            pl.BlockSpec((mt, D), lambda m: (m, 0)),
            pl.BlockSpec((D, Dh), lambda m: (0, 0)),
            pl.BlockSpec((1, Dh), lambda m: (0, 0)),
            pl.BlockSpec((Dh, E), lambda m: (0, 0)),
            pl.BlockSpec((1, E), lambda m: (0, 0)),
            pl.BlockSpec((1, 1), lambda m: (0, 0)),
        ],
        out_specs=[
            pl.BlockSpec((mt, EP), lambda m: (m, 0)),
            pl.BlockSpec((mt, E), lambda m: (m, 0)),
        ],
        out_shape=[
            jax.ShapeDtypeStruct((mc, EP), jnp.float32),
            jax.ShapeDtypeStruct((mc, E), jnp.float32),
        ],
        compiler_params=pltpu.CompilerParams(
            dimension_semantics=("arbitrary",),
        ),
    )(xf, W1, b1r, W2, b2r, tr)


TPB = 128


def _route_body(wpad_hbm, m_hbm, i_hbm, u_hbm,
                slab, mbuf, ibuf, ubuf, *, tpw, e):
    wid = lax.axis_index("s") * 2 + lax.axis_index("c")
    lanes = lax.iota(jnp.int32, L)
    zero = jnp.zeros((L,), jnp.float32)
    usage = (zero, zero, zero, zero)
    for blk in range(tpw // TPB):
        usage = _route_block(wpad_hbm, m_hbm, i_hbm,
                             slab, mbuf, ibuf,
                             wid * tpw + blk * TPB, lanes, usage, e=e)
    for c in range(4):
        ubuf[pl.ds(c * L, L)] = usage[c]
    pltpu.sync_copy(ubuf, u_hbm.at[wid])


def _route_block(wpad_hbm, m_hbm, i_hbm,
                 slab, mbuf, ibuf, base, lanes, usage0, *, e):
    pltpu.sync_copy(wpad_hbm.at[pl.ds(base, TPB)], slab)
    n_groups = TPB // L

    def group(g, usage):
        tok0 = g * L
        rows = lanes + tok0

        tkey = [jnp.full((L,), -1.0, jnp.float32) for _ in range(K)]
        tidx = [jnp.zeros((L,), jnp.int32) for _ in range(K)]
        for ex in range(e):
            xv = plsc.load_gather(slab, [rows, jnp.full((L,), ex, jnp.int32)])
            vi = jnp.full((L,), ex, jnp.int32)
            for j in range(K):
                hi = jnp.maximum(xv, tkey[j])
                lo = jnp.minimum(xv, tkey[j])
                m = xv > tkey[j]
                ni = jnp.where(m, vi, tidx[j])
                vi = jnp.where(m, tidx[j], vi)
                tkey[j], tidx[j], xv = hi, ni, lo

        sum8 = tkey[0]
        for j in range(1, K):
            sum8 = sum8 + tkey[j]
        inv_sum8 = 1.0 / sum8

        zero = jnp.zeros((L,), jnp.float32)
        for r in range(L):
            for cc in range(e // L):
                mbuf[tok0 + r, pl.ds(cc * L, L)] = zero

        for j in range(K):
            plsc.store_scatter(ibuf, [rows, jnp.full((L,), j, jnp.int32)],
                               tidx[j])
            plsc.store_scatter(mbuf, [rows, tidx[j]], tkey[j] * inv_sum8)

        u0, u1, u2, u3 = usage
        for r in range(L):
            u0 = u0 + mbuf[tok0 + r, pl.ds(0 * L, L)]
            u1 = u1 + mbuf[tok0 + r, pl.ds(1 * L, L)]
            u2 = u2 + mbuf[tok0 + r, pl.ds(2 * L, L)]
            u3 = u3 + mbuf[tok0 + r, pl.ds(3 * L, L)]
        return (u0, u1, u2, u3)

    usage = plsc.parallel_loop(0, n_groups, 1, unroll=2, carry=usage0)(group)

    pltpu.sync_copy(mbuf, m_hbm.at[pl.ds(base, TPB)])
    pltpu.sync_copy(ibuf, i_hbm.at[pl.ds(base, TPB)])
    return usage


def _sc_route(wpad, mc, E):
    tpw = mc // NW
    mesh = plsc.VectorSubcoreMesh(core_axis_name="c", subcore_axis_name="s",
                                  num_cores=2, num_subcores=16)
    body = functools.partial(_route_body, tpw=tpw, e=E)
    f = pl.kernel(
        body,
        out_type=[
            jax.ShapeDtypeStruct((mc, E), jnp.float32),
            jax.ShapeDtypeStruct((mc, K), jnp.int32),
            jax.ShapeDtypeStruct((NW, E), jnp.float32),
        ],
        mesh=mesh,
        scratch_types=[
            pltpu.VMEM((TPB, EP), jnp.float32),
            pltpu.VMEM((TPB, E), jnp.float32),
            pltpu.VMEM((TPB, K), jnp.int32),
            pltpu.VMEM((E,), jnp.float32),
        ],
        compiler_params=pltpu.CompilerParams(needs_layout_passes=False),
    )
    return f(wpad)


def _loss_body(u_ref, loss_ref, *, e, capacity):
    usage = jnp.sum(u_ref[...], axis=0, keepdims=True)
    ideal = jnp.sum(usage) / e
    lb = jnp.mean((usage - ideal) ** 2)
    cl = jnp.mean(jnp.maximum(usage - capacity, 0.0))
    loss_ref[...] = jnp.full((1, 1), lb + cl, jnp.float32)


def _tc_loss(usage_parts, E, capacity):
    n = usage_parts.shape[0]
    return pl.pallas_call(
        functools.partial(_loss_body, e=E, capacity=capacity),
        grid=(1,),
        in_specs=[pl.BlockSpec((n, E), lambda i: (0, 0))],
        out_specs=pl.BlockSpec((1, 1), lambda i: (0, 0)),
        out_shape=jax.ShapeDtypeStruct((1, 1), jnp.float32),
    )(usage_parts)


def kernel(x, W1, b1, W2, b2, temperature):
    B, S, D = x.shape
    Dh, E = W2.shape
    M = B * S
    xf = x.reshape(M, D)
    b1r = b1.reshape(1, Dh)
    b2r = b2.reshape(1, E)
    tr = temperature.reshape(1, 1)
    capacity = float(int(1.25 * S))

    mc = M // N_CHUNKS
    weights, mask, idx, usage = [], [], [], []
    for c in range(N_CHUNKS):
        wpad, w_c = _tc_scores(xf[c * mc:(c + 1) * mc], W1, b1r, W2, b2r, tr)
        m_c, i_c, u_c = _sc_route(wpad, mc, E)
        weights.append(w_c)
        mask.append(m_c.reshape(mc, E))
        idx.append(i_c.reshape(mc, K))
        usage.append(u_c.reshape(NW, E))

    usage_parts = jnp.concatenate(usage, axis=0)
    loss = _tc_loss(usage_parts, E, capacity)

    mask_f = jnp.concatenate(mask, axis=0).reshape(B, S, E)
    weights_f = jnp.concatenate(weights, axis=0).reshape(B, S, E)
    idx_f = jnp.concatenate(idx, axis=0).reshape(B, S, K)
    return (mask_f, loss.reshape(()), weights_f, idx_f)


# speedup vs baseline: 1.0485x; 1.0485x over previous
import functools

import jax
import jax.numpy as jnp
from jax import lax
from jax.experimental import pallas as pl
from jax.experimental.pallas import tpu as pltpu
from jax.experimental.pallas import tpu_sc as plsc

M_TILE = 1024
N_CHUNKS = 1
K = 8
NW = 32
L = 16
EP = 128


def _scores_body(x_ref, w1_ref, b1_ref, w2_ref, b2_ref, temp_ref,
                 wpad_ref, w_ref):
    hid = jnp.dot(x_ref[...], w1_ref[...], preferred_element_type=jnp.float32)
    hid = hid + b1_ref[...]
    hid = hid * 0.5 * (1.0 + jax.lax.erf(hid * 0.7071067811865476))
    scores = jnp.dot(hid, w2_ref[...], preferred_element_type=jnp.float32)
    logits = (scores + b2_ref[...]) / temp_ref[0, 0]
    lmax = jnp.max(logits, axis=-1, keepdims=True)
    ex = jnp.exp(logits - lmax)
    w = ex / jnp.sum(ex, axis=-1, keepdims=True)
    w_ref[...] = w
    mt = w.shape[0]
    pad = jnp.zeros((mt, EP - w.shape[1]), jnp.float32)
    wpad_ref[...] = jnp.concatenate([w, pad], axis=1)


def _tc_scores(xf, W1, b1r, W2, b2r, tr):
    mc, D = xf.shape
    Dh, E = W2.shape
    mt = min(M_TILE, mc)
    return pl.pallas_call(
        _scores_body,
        grid=(mc // mt,),
        in_specs=[
            pl.BlockSpec((mt, D), lambda m: (m, 0)),
            pl.BlockSpec((D, Dh), lambda m: (0, 0)),
            pl.BlockSpec((1, Dh), lambda m: (0, 0)),
            pl.BlockSpec((Dh, E), lambda m: (0, 0)),
            pl.BlockSpec((1, E), lambda m: (0, 0)),
            pl.BlockSpec((1, 1), lambda m: (0, 0)),
        ],
        out_specs=[
            pl.BlockSpec((mt, EP), lambda m: (m, 0)),
            pl.BlockSpec((mt, E), lambda m: (m, 0)),
        ],
        out_shape=[
            jax.ShapeDtypeStruct((mc, EP), jnp.float32),
            jax.ShapeDtypeStruct((mc, E), jnp.float32),
        ],
        compiler_params=pltpu.CompilerParams(
            dimension_semantics=("arbitrary",),
        ),
    )(xf, W1, b1r, W2, b2r, tr)


TPB = 128


def _route_body(wpad_hbm, m_hbm, i_hbm, u_hbm,
                slab, mbuf, ibuf, ubuf, *, tpw, e):
    wid = lax.axis_index("s") * 2 + lax.axis_index("c")
    lanes = lax.iota(jnp.int32, L)
    zero = jnp.zeros((L,), jnp.float32)
    usage = (zero, zero, zero, zero)
    for blk in range(tpw // TPB):
        usage = _route_block(wpad_hbm, m_hbm, i_hbm,
                             slab, mbuf, ibuf,
                             wid * tpw + blk * TPB, lanes, usage, e=e)
    for c in range(4):
        ubuf[pl.ds(c * L, L)] = usage[c]
    pltpu.sync_copy(ubuf, u_hbm.at[wid])


def _route_block(wpad_hbm, m_hbm, i_hbm,
                 slab, mbuf, ibuf, base, lanes, usage0, *, e):
    pltpu.sync_copy(wpad_hbm.at[pl.ds(base, TPB)], slab)
    n_groups = TPB // L

    def group(g, usage):
        tok0 = g * L
        rows = lanes + tok0

        tkey = [jnp.full((L,), -1.0, jnp.float32) for _ in range(K)]
        tidx = [jnp.zeros((L,), jnp.int32) for _ in range(K)]
        for ex in range(e):
            xv = plsc.load_gather(slab, [rows, jnp.full((L,), ex, jnp.int32)])
            vi = jnp.full((L,), ex, jnp.int32)
            for j in range(K):
                hi = jnp.maximum(xv, tkey[j])
                lo = jnp.minimum(xv, tkey[j])
                m = xv > tkey[j]
                ni = jnp.where(m, vi, tidx[j])
                vi = jnp.where(m, tidx[j], vi)
                tkey[j], tidx[j], xv = hi, ni, lo

        sum8 = tkey[0]
        for j in range(1, K):
            sum8 = sum8 + tkey[j]
        inv_sum8 = 1.0 / sum8

        zero = jnp.zeros((L,), jnp.float32)
        for r in range(L):
            for cc in range(e // L):
                mbuf[tok0 + r, pl.ds(cc * L, L)] = zero

        for j in range(K):
            plsc.store_scatter(ibuf, [rows, jnp.full((L,), j, jnp.int32)],
                               tidx[j])
            plsc.store_scatter(mbuf, [rows, tidx[j]], tkey[j] * inv_sum8)

        u0, u1, u2, u3 = usage
        for r in range(L):
            u0 = u0 + mbuf[tok0 + r, pl.ds(0 * L, L)]
            u1 = u1 + mbuf[tok0 + r, pl.ds(1 * L, L)]
            u2 = u2 + mbuf[tok0 + r, pl.ds(2 * L, L)]
            u3 = u3 + mbuf[tok0 + r, pl.ds(3 * L, L)]
        return (u0, u1, u2, u3)

    usage = plsc.parallel_loop(0, n_groups, 1, unroll=1, carry=usage0)(group)

    pltpu.sync_copy(mbuf, m_hbm.at[pl.ds(base, TPB)])
    pltpu.sync_copy(ibuf, i_hbm.at[pl.ds(base, TPB)])
    return usage


def _sc_route(wpad, mc, E):
    tpw = mc // NW
    mesh = plsc.VectorSubcoreMesh(core_axis_name="c", subcore_axis_name="s",
                                  num_cores=2, num_subcores=16)
    body = functools.partial(_route_body, tpw=tpw, e=E)
    f = pl.kernel(
        body,
        out_type=[
            jax.ShapeDtypeStruct((mc, E), jnp.float32),
            jax.ShapeDtypeStruct((mc, K), jnp.int32),
            jax.ShapeDtypeStruct((NW, E), jnp.float32),
        ],
        mesh=mesh,
        scratch_types=[
            pltpu.VMEM((TPB, EP), jnp.float32),
            pltpu.VMEM((TPB, E), jnp.float32),
            pltpu.VMEM((TPB, K), jnp.int32),
            pltpu.VMEM((E,), jnp.float32),
        ],
        compiler_params=pltpu.CompilerParams(needs_layout_passes=False),
    )
    return f(wpad)


def _loss_body(u_ref, loss_ref, *, e, capacity):
    usage = jnp.sum(u_ref[...], axis=0, keepdims=True)
    ideal = jnp.sum(usage) / e
    lb = jnp.mean((usage - ideal) ** 2)
    cl = jnp.mean(jnp.maximum(usage - capacity, 0.0))
    loss_ref[...] = jnp.full((1, 1), lb + cl, jnp.float32)


def _tc_loss(usage_parts, E, capacity):
    n = usage_parts.shape[0]
    return pl.pallas_call(
        functools.partial(_loss_body, e=E, capacity=capacity),
        grid=(1,),
        in_specs=[pl.BlockSpec((n, E), lambda i: (0, 0))],
        out_specs=pl.BlockSpec((1, 1), lambda i: (0, 0)),
        out_shape=jax.ShapeDtypeStruct((1, 1), jnp.float32),
    )(usage_parts)


def kernel(x, W1, b1, W2, b2, temperature):
    B, S, D = x.shape
    Dh, E = W2.shape
    M = B * S
    xf = x.reshape(M, D)
    b1r = b1.reshape(1, Dh)
    b2r = b2.reshape(1, E)
    tr = temperature.reshape(1, 1)
    capacity = float(int(1.25 * S))

    mc = M // N_CHUNKS
    weights, mask, idx, usage = [], [], [], []
    for c in range(N_CHUNKS):
        wpad, w_c = _tc_scores(xf[c * mc:(c + 1) * mc], W1, b1r, W2, b2r, tr)
        m_c, i_c, u_c = _sc_route(wpad, mc, E)
        weights.append(w_c)
        mask.append(m_c.reshape(mc, E))
        idx.append(i_c.reshape(mc, K))
        usage.append(u_c.reshape(NW, E))

    usage_parts = jnp.concatenate(usage, axis=0)
    loss = _tc_loss(usage_parts, E, capacity)

    mask_f = jnp.concatenate(mask, axis=0).reshape(B, S, E)
    weights_f = jnp.concatenate(weights, axis=0).reshape(B, S, E)
    idx_f = jnp.concatenate(idx, axis=0).reshape(B, S, K)
    return (mask_f, loss.reshape(()), weights_f, idx_f)
